# baseline (speedup 1.0000x reference)
import functools

import jax
import jax.numpy as jnp
from jax import lax
from jax.experimental import pallas as pl
from jax.experimental.pallas import tpu as pltpu
from jax.experimental.pallas import tpu_sc as plsc

N = 50000
E = 800000
D = 128
DH = 64
DE = 16

NCORE = 2
NSUB = 16
NWORK = NCORE * NSUB

CH = 512
SB = 128
NSB = CH // SB
EPW = 25088
EPAD = EPW * NWORK
TRIPS = EPW // CH
EPT = EPAD // NSUB
STRIPS = EPT // CH
IDXR = EPAD // SB

NH = N // 2
ACC = 25088
NZCH = ACC // SB
NWCH = 196

TE = 4096
TN = 2000


def _silu(v):
    return v * jax.nn.sigmoid(v)


def _sc_mesh():
    return plsc.VectorSubcoreMesh(
        core_axis_name="c", subcore_axis_name="s",
        num_cores=NCORE, num_subcores=NSUB)


def _rel_gather(pos16, s2d, d2d):
    def body(p_hbm, s_hbm, d_hbm, rel_hbm, sidx, didx, pbuf, qbuf,
             semi, semg):
        c = lax.axis_index("c")
        s = lax.axis_index("s")
        w = s * NCORE + c

        def trip(t, cy):
            g = w * EPW + t * CH
            r = w * (EPW // SB) + t * NSB
            di = pltpu.async_copy(s_hbm.at[pl.ds(r, NSB)], sidx, semi)
            dj = pltpu.async_copy(d_hbm.at[pl.ds(r, NSB)], didx, semi)
            di.wait()
            dj.wait()
            ds_ = []
            for j in range(NSB):
                ds_.append(pltpu.async_copy(p_hbm.at[sidx.at[j]],
                                            pbuf.at[pl.ds(j * SB, SB)],
                                            semg))
                ds_.append(pltpu.async_copy(p_hbm.at[didx.at[j]],
                                            qbuf.at[pl.ds(j * SB, SB)],
                                            semg))
            for dd in ds_:
                dd.wait()

            def sub_row(rr, cy2):
                pbuf[rr, pl.ds(0, 16)] = (pbuf[rr, pl.ds(0, 16)]
                                          - qbuf[rr, pl.ds(0, 16)])
                return cy2

            lax.fori_loop(0, CH, sub_row, 0)
            pltpu.sync_copy(pbuf, rel_hbm.at[pl.ds(g, CH)])
            return cy

        lax.fori_loop(0, TRIPS, trip, 0)

    f = pl.kernel(
        body,
        out_type=jax.ShapeDtypeStruct((EPAD, 16), jnp.float32),
        mesh=_sc_mesh(),
        compiler_params=pltpu.CompilerParams(use_tc_tiling_on_sc=False),
        scratch_types=[
            pltpu.VMEM((NSB, SB), jnp.int32),
            pltpu.VMEM((NSB, SB), jnp.int32),
            pltpu.VMEM((CH, 16), jnp.float32),
            pltpu.VMEM((CH, 16), jnp.float32),
            pltpu.SemaphoreType.DMA,
            pltpu.SemaphoreType.DMA,
        ],
        name="rel_gather")
    return f(pos16, s2d, d2d)


def _z_gather(a_tab, b_tab, s2d, d2d):
    def body(a_hbm, b_hbm, s_hbm, d_hbm, z_hbm, sidx, didx, abuf, bbuf,
             semi, semg):
        c = lax.axis_index("c")
        s = lax.axis_index("s")
        w = s * NCORE + c

        def trip(t, cy):
            g = w * EPW + t * CH
            r = w * (EPW // SB) + t * NSB
            di = pltpu.async_copy(s_hbm.at[pl.ds(r, NSB)], sidx, semi)
            dj = pltpu.async_copy(d_hbm.at[pl.ds(r, NSB)], didx, semi)
            di.wait()
            dj.wait()
            ds_ = []
            for j in range(NSB):
                ds_.append(pltpu.async_copy(a_hbm.at[sidx.at[j]],
                                            abuf.at[pl.ds(j * SB, SB)],
                                            semg))
                ds_.append(pltpu.async_copy(b_hbm.at[didx.at[j]],
                                            bbuf.at[pl.ds(j * SB, SB)],
                                            semg))
            for dd in ds_:
                dd.wait()

            def add_row(rr, cy2):
                for j4 in range(DH // 32):
                    abuf[rr, pl.ds(j4 * 32, 32)] = (
                        abuf[rr, pl.ds(j4 * 32, 32)]
                        + bbuf[rr, pl.ds(j4 * 32, 32)])
                return cy2

            lax.fori_loop(0, CH, add_row, 0)
            pltpu.sync_copy(abuf, z_hbm.at[pl.ds(g, CH)])
            return cy

        lax.fori_loop(0, TRIPS, trip, 0)

    f = pl.kernel(
        body,
        out_type=jax.ShapeDtypeStruct((EPAD, DH), jnp.bfloat16),
        mesh=_sc_mesh(),
        compiler_params=pltpu.CompilerParams(use_tc_tiling_on_sc=False),
        scratch_types=[
            pltpu.VMEM((NSB, SB), jnp.int32),
            pltpu.VMEM((NSB, SB), jnp.int32),
            pltpu.VMEM((CH, DH), jnp.bfloat16),
            pltpu.VMEM((CH, DH), jnp.bfloat16),
            pltpu.SemaphoreType.DMA,
            pltpu.SemaphoreType.DMA,
        ],
        name="z_gather")
    return f(a_tab, b_tab, s2d, d2d)


def _make_scatter(width, name):
    def body(v_hbm, d_hbm, z_hbm, out_hbm, didx, vbuf, acc, semi, sems):
        c = lax.axis_index("c")
        s = lax.axis_index("s")
        base = c * NH
        dump = NH + 4 * s

        for j in range(13):
            cidx = s + NSUB * j

            @pl.when(cidx < NZCH)
            def _():
                pltpu.sync_copy(z_hbm, acc.at[pl.ds(cidx * SB, SB)])

        plsc.subcore_barrier()

        def trip(t, cy):
            g = s * EPT + t * CH
            r = s * (EPT // SB) + t * NSB
            di = pltpu.async_copy(d_hbm.at[pl.ds(r, NSB)], didx, semi)
            dv = pltpu.async_copy(v_hbm.at[pl.ds(g, CH)], vbuf, semi)
            di.wait()
            dv.wait()
            for j in range(NSB):
                for v in range(SB // 16):
                    dv = didx[j, pl.ds(v * 16, 16)]
                    loc = dv - base
                    ok = (loc >= 0) & (loc < NH)
                    didx[j, pl.ds(v * 16, 16)] = jnp.where(ok, loc, dump)
            ds_ = []
            for j in range(NSB):
                ds_.append(pltpu.async_copy(vbuf.at[pl.ds(j * SB, SB)],
                                            acc.at[didx.at[j]], sems,
                                            add=True))
            for dd in ds_:
                dd.wait()
            return cy

        lax.fori_loop(0, STRIPS, trip, 0)
        plsc.subcore_barrier()

        for j in range(13):
            cidx = s + NSUB * j

            @pl.when(cidx < NWCH - 1)
            def _():
                pltpu.sync_copy(acc.at[pl.ds(cidx * SB, SB)],
                                out_hbm.at[pl.ds(base + cidx * SB, SB)])

            @pl.when(cidx == NWCH - 1)
            def _():
                pltpu.sync_copy(acc.at[pl.ds(cidx * SB, 40)],
                                out_hbm.at[pl.ds(base + cidx * SB, 40)])

    def call(v, d2d, z):
        f = pl.kernel(
            body,
            out_type=jax.ShapeDtypeStruct((N, width), jnp.float32),
            mesh=_sc_mesh(),
            compiler_params=pltpu.CompilerParams(use_tc_tiling_on_sc=False),
            scratch_types=[
                pltpu.VMEM((NSB, SB), jnp.int32),
                pltpu.VMEM((CH, width), jnp.float32),
                pltpu.VMEM_SHARED((ACC, width), jnp.float32),
                pltpu.SemaphoreType.DMA,
                pltpu.SemaphoreType.DMA,
            ],
            name=name)
        return f(v, d2d, z)

    return call


_scatter_lo = _make_scatter(40, "seg_scatter_lo")
_scatter_hi = _make_scatter(32, "seg_scatter_hi")


def _prep(h, ws, wd, be):
    def body(h_ref, ws_ref, wd_ref, be_ref, a_ref, b_ref):
        h_ = h_ref[...]
        a_ref[...] = (jnp.dot(h_, ws_ref[...],
                              preferred_element_type=jnp.float32)
                      + be_ref[...]).astype(jnp.bfloat16)
        b_ref[...] = jnp.dot(h_, wd_ref[...],
                             preferred_element_type=jnp.float32
                             ).astype(jnp.bfloat16)

    grid = (N // TN,)
    big = pl.BlockSpec((TN, DH), lambda i: (i, 0))
    wsp = pl.BlockSpec((DH, DH), lambda i: (0, 0))
    bsp = pl.BlockSpec((1, DH), lambda i: (0, 0))
    return pl.pallas_call(
        body,
        grid=grid,
        in_specs=[big, wsp, wsp, bsp],
        out_specs=[big, big],
        out_shape=[jax.ShapeDtypeStruct((N, DH), jnp.bfloat16)] * 2,
    )(h, ws, wd, be)


def _edge_mlp(z, rel, ea, wea, wdist, we2, be2, wp1, bp1, wp2, bp2):
    def body(z_ref, r_ref, e_ref, wea_ref, wd_ref, we2_ref, be2_ref,
             wp1_ref, bp1_ref, wp2_ref, bp2_ref, lo_ref, hi_ref):
        rel_ = r_ref[...]
        dist = jnp.sum(rel_ * rel_, axis=1, keepdims=True)
        pre1 = (z_ref[...].astype(jnp.float32) + dist * wd_ref[...]
                + jnp.dot(e_ref[...], wea_ref[...],
                          preferred_element_type=jnp.float32))
        m1 = _silu(pre1)
        m2 = _silu(jnp.dot(m1, we2_ref[...],
                           preferred_element_type=jnp.float32) + be2_ref[...])
        p = _silu(jnp.dot(m2, wp1_ref[...],
                          preferred_element_type=jnp.float32) + bp1_ref[...])
        coef = jnp.sum(p * wp2_ref[...], axis=1, keepdims=True) + bp2_ref[...]
        lo_ref[...] = jnp.concatenate(
            [m2[:, 0:32], rel_[:, 0:4] * coef,
             jnp.ones((TE, 4), jnp.float32)], axis=1)
        hi_ref[...] = m2[:, 32:64]

    grid = (EPAD // TE,)
    zsp = pl.BlockSpec((TE, DH), lambda i: (i, 0))
    rsp = pl.BlockSpec((TE, 16), lambda i: (i, 0))
    esp = pl.BlockSpec((TE, DE), lambda i: (i, 0))
    w16 = pl.BlockSpec((DE, DH), lambda i: (0, 0))
    w64 = pl.BlockSpec((DH, DH), lambda i: (0, 0))
    row = pl.BlockSpec((1, DH), lambda i: (0, 0))
    sca = pl.BlockSpec((1, 1), lambda i: (0, 0))
    losp = pl.BlockSpec((TE, 40), lambda i: (i, 0))
    hisp = pl.BlockSpec((TE, 32), lambda i: (i, 0))
    return pl.pallas_call(
        body,
        grid=grid,
        in_specs=[zsp, rsp, esp, w16, row, w64, row, w64, row, row, sca],
        out_specs=[losp, hisp],
        out_shape=[jax.ShapeDtypeStruct((EPAD, 40), jnp.float32),
                   jax.ShapeDtypeStruct((EPAD, 32), jnp.float32)],
    )(z, rel, ea, wea, wdist, we2, be2, wp1, bp1, wp2, bp2)


def _node0(h, cadd, agglo, agghi, wn1h, wn1lo, wn1hi, bn1, wn2, bn2,
           wes, wed, ben):
    def body(h_ref, c_ref, glo_ref, ghi_ref, w1h_ref, w1lo_ref, w1hi_ref,
             b1_ref, w2_ref, b2_ref,
             wes_ref, wed_ref, ben_ref, y_ref, a_ref, b_ref):
        t = _silu(jnp.dot(h_ref[...], w1h_ref[...],
                          preferred_element_type=jnp.float32)
                  + jnp.dot(glo_ref[...][:, 0:32], w1lo_ref[...],
                            preferred_element_type=jnp.float32)
                  + jnp.dot(ghi_ref[...], w1hi_ref[...],
                            preferred_element_type=jnp.float32)
                  + b1_ref[...])
        d = jnp.dot(t, w2_ref[...],
                    preferred_element_type=jnp.float32) + b2_ref[...]
        y = c_ref[...] + d
        y_ref[...] = y
        a_ref[...] = (jnp.dot(y, wes_ref[...],
                              preferred_element_type=jnp.float32)
                      + ben_ref[...]).astype(jnp.bfloat16)
        b_ref[...] = jnp.dot(y, wed_ref[...],
                             preferred_element_type=jnp.float32
                             ).astype(jnp.bfloat16)

    grid = (N // TN,)
    big = pl.BlockSpec((TN, DH), lambda i: (i, 0))
    glo = pl.BlockSpec((TN, 40), lambda i: (i, 0))
    ghi = pl.BlockSpec((TN, 32), lambda i: (i, 0))
    w64 = pl.BlockSpec((DH, DH), lambda i: (0, 0))
    w32 = pl.BlockSpec((32, DH), lambda i: (0, 0))
    row = pl.BlockSpec((1, DH), lambda i: (0, 0))
    return pl.pallas_call(
        body,
        grid=grid,
        in_specs=[big, big, glo, ghi, w64, w32, w32, row, w64, row,
                  w64, w64, row],
        out_specs=[big, big, big],
        out_shape=[jax.ShapeDtypeStruct((N, DH), jnp.float32),
                   jax.ShapeDtypeStruct((N, DH), jnp.bfloat16),
                   jax.ShapeDtypeStruct((N, DH), jnp.bfloat16)],
    )(h, cadd, agglo, agghi, wn1h, wn1lo, wn1hi, bn1, wn2, bn2,
      wes, wed, ben)


def _node1(h, cadd, agglo, agghi, wn1h, wn1lo, wn1hi, bn1, wn2, bn2,
           tlo0, pa8, pb8):
    def body(h_ref, c_ref, glo_ref, ghi_ref, w1h_ref, w1lo_ref, w1hi_ref,
             b1_ref, w2_ref, b2_ref,
             t0_ref, pa_ref, pb_ref, y_ref, pc_ref):
        glo = glo_ref[...]
        t = _silu(jnp.dot(h_ref[...], w1h_ref[...],
                          preferred_element_type=jnp.float32)
                  + jnp.dot(glo[:, 0:32], w1lo_ref[...],
                            preferred_element_type=jnp.float32)
                  + jnp.dot(ghi_ref[...], w1hi_ref[...],
                            preferred_element_type=jnp.float32)
                  + b1_ref[...])
        d = jnp.dot(t, w2_ref[...],
                    preferred_element_type=jnp.float32) + b2_ref[...]
        y_ref[...] = c_ref[...] + d
        t0 = t0_ref[...][:, 32:40]
        t1 = glo[:, 32:40]
        deg = t0[:, 4:5]
        recip = 1.0 / jnp.maximum(deg, 1.0)
        pc_ref[...] = (0.5 * (pa_ref[...] + pb_ref[...])
                       + 0.25 * (t0 + t1) * recip)

    grid = (N // TN,)
    big = pl.BlockSpec((TN, DH), lambda i: (i, 0))
    glo = pl.BlockSpec((TN, 40), lambda i: (i, 0))
    ghi = pl.BlockSpec((TN, 32), lambda i: (i, 0))
    w64 = pl.BlockSpec((DH, DH), lambda i: (0, 0))
    w32 = pl.BlockSpec((32, DH), lambda i: (0, 0))
    row = pl.BlockSpec((1, DH), lambda i: (0, 0))
    td8 = pl.BlockSpec((TN, 8), lambda i: (i, 0))
    return pl.pallas_call(
        body,
        grid=grid,
        in_specs=[big, big, glo, ghi, w64, w32, w32, row, w64, row,
                  glo, td8, td8],
        out_specs=[big, td8],
        out_shape=[jax.ShapeDtypeStruct((N, DH), jnp.float32),
                   jax.ShapeDtypeStruct((N, 8), jnp.float32)],
    )(h, cadd, agglo, agghi, wn1h, wn1lo, wn1hi, bn1, wn2, bn2,
      tlo0, pa8, pb8)


def kernel(x, pos_a, pos_b, edge_index, edge_attr, We1, be1, We2, be2,
           Wp1, bp1, Wp2, bp2, Wn1, bn1, Wn2, bn2):
    f32 = jnp.float32
    src = edge_index[0]
    dst = edge_index[1]
    npad = EPAD - E

    pad_idx = (jnp.arange(npad, dtype=jnp.int32) * 97) % N
    src_g = jnp.concatenate([src, pad_idx]).reshape(IDXR, SB)
    dst_g = jnp.concatenate([dst, pad_idx]).reshape(IDXR, SB)
    dst_s = jnp.concatenate(
        [dst, jnp.full((npad,), N, jnp.int32)]).reshape(IDXR, SB)

    ea_p = jnp.pad(edge_attr, ((0, npad), (0, 0)))
    pos16 = jnp.pad(pos_a, ((0, 0), (0, 13)))
    pa8 = jnp.pad(pos_a, ((0, 0), (0, 5)))
    pb8 = jnp.pad(pos_b, ((0, 0), (0, 5)))
    z40 = jnp.zeros((SB, 40), f32)
    z32 = jnp.zeros((SB, 32), f32)

    c0 = x[:, :DH]
    c1 = x[:, DH:]

    def wsplit(i):
        w1 = We1[i]
        return (w1[0:DH], w1[DH:2 * DH], w1[2 * DH].reshape(1, DH),
                w1[2 * DH + 1:].reshape(DE, DH), be1[i].reshape(1, DH))

    ws0, wd0, wdist0, wea0, be10 = wsplit(0)
    ws1, wd1, wdist1, wea1, be11 = wsplit(1)

    rel = _rel_gather(pos16, src_g, dst_g)

    a0, b0 = _prep(c1, ws0, wd0, be10)
    z0 = _z_gather(a0, b0, src_g, dst_g)
    mlo0, mhi0 = _edge_mlp(z0, rel, ea_p, wea0, wdist0,
                         We2[0], be2[0].reshape(1, DH),
                         Wp1[0], bp1[0].reshape(1, DH),
                         Wp2[0].reshape(1, DH), bp2[0].reshape(1, 1))
    agglo0 = _scatter_lo(mlo0, dst_s, z40)
    agghi0 = _scatter_hi(mhi0, dst_s, z32)
    y0, a1, b1 = _node0(c1, c0, agglo0, agghi0,
                        Wn1[0][0:DH], Wn1[0][DH:DH + 32], Wn1[0][DH + 32:],
                        bn1[0].reshape(1, DH),
                        Wn2[0], bn2[0].reshape(1, DH), ws1, wd1, be11)

    z1 = _z_gather(a1, b1, src_g, dst_g)
    mlo1, mhi1 = _edge_mlp(z1, rel, ea_p, wea1, wdist1,
                         We2[1], be2[1].reshape(1, DH),
                         Wp1[1], bp1[1].reshape(1, DH),
                         Wp2[1].reshape(1, DH), bp2[1].reshape(1, 1))
    agglo1 = _scatter_lo(mlo1, dst_s, z40)
    agghi1 = _scatter_hi(mhi1, dst_s, z32)
    y1, pc8 = _node1(y0, c1, agglo1, agghi1,
                     Wn1[1][0:DH], Wn1[1][DH:DH + 32], Wn1[1][DH + 32:],
                     bn1[1].reshape(1, DH),
                     Wn2[1], bn2[1].reshape(1, DH),
                     agglo0, pa8, pb8)

    y = jnp.concatenate([y0, y1], axis=-1)
    pos_c = pc8[:, :3]
    return (y, pos_c, pos_a)


# speedup vs baseline: 1.0394x; 1.0394x over previous
import functools

import jax
import jax.numpy as jnp
from jax import lax
from jax.experimental import pallas as pl
from jax.experimental.pallas import tpu as pltpu
from jax.experimental.pallas import tpu_sc as plsc

N = 50000
E = 800000
D = 128
DH = 64
DE = 16

NCORE = 2
NSUB = 16
NWORK = NCORE * NSUB

CH = 512
SB = 128
NSB = CH // SB
EPW = 25088
EPAD = EPW * NWORK
TRIPS = EPW // CH
EPT = EPAD // NSUB
STRIPS = EPT // CH
IDXR = EPAD // SB

NH = N // 2
ACC = 25088
NZCH = ACC // SB
NWCH = 196

TE = 4096
TN = 2000


def _silu(v):
    return v * jax.nn.sigmoid(v)


def _sc_mesh():
    return plsc.VectorSubcoreMesh(
        core_axis_name="c", subcore_axis_name="s",
        num_cores=NCORE, num_subcores=NSUB)


def _rel_gather(pos16, s2d, d2d):
    def body(p_hbm, s_hbm, d_hbm, rel_hbm, sidx, didx, pbuf, qbuf,
             semi, semg):
        c = lax.axis_index("c")
        s = lax.axis_index("s")
        w = s * NCORE + c

        def trip(t, cy):
            g = w * EPW + t * CH
            r = w * (EPW // SB) + t * NSB
            di = pltpu.async_copy(s_hbm.at[pl.ds(r, NSB)], sidx, semi)
            dj = pltpu.async_copy(d_hbm.at[pl.ds(r, NSB)], didx, semi)
            di.wait()
            dj.wait()
            ds_ = []
            for j in range(NSB):
                ds_.append(pltpu.async_copy(p_hbm.at[sidx.at[j]],
                                            pbuf.at[pl.ds(j * SB, SB)],
                                            semg))
                ds_.append(pltpu.async_copy(p_hbm.at[didx.at[j]],
                                            qbuf.at[pl.ds(j * SB, SB)],
                                            semg))
            for dd in ds_:
                dd.wait()

            def sub_row(rr, cy2):
                pbuf[rr, pl.ds(0, 16)] = (pbuf[rr, pl.ds(0, 16)]
                                          - qbuf[rr, pl.ds(0, 16)])
                return cy2

            lax.fori_loop(0, CH, sub_row, 0)
            pltpu.sync_copy(pbuf, rel_hbm.at[pl.ds(g, CH)])
            return cy

        lax.fori_loop(0, TRIPS, trip, 0)

    f = pl.kernel(
        body,
        out_type=jax.ShapeDtypeStruct((EPAD, 16), jnp.float32),
        mesh=_sc_mesh(),
        compiler_params=pltpu.CompilerParams(use_tc_tiling_on_sc=False),
        scratch_types=[
            pltpu.VMEM((NSB, SB), jnp.int32),
            pltpu.VMEM((NSB, SB), jnp.int32),
            pltpu.VMEM((CH, 16), jnp.float32),
            pltpu.VMEM((CH, 16), jnp.float32),
            pltpu.SemaphoreType.DMA,
            pltpu.SemaphoreType.DMA,
        ],
        name="rel_gather")
    return f(pos16, s2d, d2d)


def _z_gather(a_tab, b_tab, s2d, d2d):
    def body(a_hbm, b_hbm, s_hbm, d_hbm, z_hbm, sidx, didx, abuf, bbuf,
             semi, semg):
        c = lax.axis_index("c")
        s = lax.axis_index("s")
        w = s * NCORE + c

        def trip(t, cy):
            g = w * EPW + t * CH
            r = w * (EPW // SB) + t * NSB
            di = pltpu.async_copy(s_hbm.at[pl.ds(r, NSB)], sidx, semi)
            dj = pltpu.async_copy(d_hbm.at[pl.ds(r, NSB)], didx, semi)
            di.wait()
            dj.wait()
            ds_ = []
            for j in range(NSB):
                ds_.append(pltpu.async_copy(a_hbm.at[sidx.at[j]],
                                            abuf.at[pl.ds(j * SB, SB)],
                                            semg))
                ds_.append(pltpu.async_copy(b_hbm.at[didx.at[j]],
                                            bbuf.at[pl.ds(j * SB, SB)],
                                            semg))
            for dd in ds_:
                dd.wait()

            def add_row(rr, cy2):
                for j4 in range(DH // 16):
                    abuf[rr, pl.ds(j4 * 16, 16)] = (
                        abuf[rr, pl.ds(j4 * 16, 16)]
                        + bbuf[rr, pl.ds(j4 * 16, 16)])
                return cy2

            lax.fori_loop(0, CH, add_row, 0)
            pltpu.sync_copy(abuf, z_hbm.at[pl.ds(g, CH)])
            return cy

        lax.fori_loop(0, TRIPS, trip, 0)

    f = pl.kernel(
        body,
        out_type=jax.ShapeDtypeStruct((EPAD, DH), jnp.float32),
        mesh=_sc_mesh(),
        compiler_params=pltpu.CompilerParams(use_tc_tiling_on_sc=False),
        scratch_types=[
            pltpu.VMEM((NSB, SB), jnp.int32),
            pltpu.VMEM((NSB, SB), jnp.int32),
            pltpu.VMEM((CH, DH), jnp.float32),
            pltpu.VMEM((CH, DH), jnp.float32),
            pltpu.SemaphoreType.DMA,
            pltpu.SemaphoreType.DMA,
        ],
        name="z_gather")
    return f(a_tab, b_tab, s2d, d2d)


def _make_scatter(width, name):
    def body(v_hbm, d_hbm, z_hbm, out_hbm, didx, vbuf, acc, semi, sems):
        c = lax.axis_index("c")
        s = lax.axis_index("s")
        base = c * NH
        dump = NH + 4 * s

        for j in range(13):
            cidx = s + NSUB * j

            @pl.when(cidx < NZCH)
            def _():
                pltpu.sync_copy(z_hbm, acc.at[pl.ds(cidx * SB, SB)])

        plsc.subcore_barrier()

        def trip(t, cy):
            g = s * EPT + t * CH
            r = s * (EPT // SB) + t * NSB
            di = pltpu.async_copy(d_hbm.at[pl.ds(r, NSB)], didx, semi)
            dv = pltpu.async_copy(v_hbm.at[pl.ds(g, CH)], vbuf, semi)
            di.wait()
            dv.wait()
            for j in range(NSB):
                for v in range(SB // 16):
                    dv = didx[j, pl.ds(v * 16, 16)]
                    loc = dv - base
                    ok = (loc >= 0) & (loc < NH)
                    didx[j, pl.ds(v * 16, 16)] = jnp.where(ok, loc, dump)
            ds_ = []
            for j in range(NSB):
                ds_.append(pltpu.async_copy(vbuf.at[pl.ds(j * SB, SB)],
                                            acc.at[didx.at[j]], sems,
                                            add=True))
            for dd in ds_:
                dd.wait()
            return cy

        lax.fori_loop(0, STRIPS, trip, 0)
        plsc.subcore_barrier()

        for j in range(13):
            cidx = s + NSUB * j

            @pl.when(cidx < NWCH - 1)
            def _():
                pltpu.sync_copy(acc.at[pl.ds(cidx * SB, SB)],
                                out_hbm.at[pl.ds(base + cidx * SB, SB)])

            @pl.when(cidx == NWCH - 1)
            def _():
                pltpu.sync_copy(acc.at[pl.ds(cidx * SB, 40)],
                                out_hbm.at[pl.ds(base + cidx * SB, 40)])

    def call(v, d2d, z):
        f = pl.kernel(
            body,
            out_type=jax.ShapeDtypeStruct((N, width), jnp.float32),
            mesh=_sc_mesh(),
            compiler_params=pltpu.CompilerParams(use_tc_tiling_on_sc=False),
            scratch_types=[
                pltpu.VMEM((NSB, SB), jnp.int32),
                pltpu.VMEM((CH, width), jnp.float32),
                pltpu.VMEM_SHARED((ACC, width), jnp.float32),
                pltpu.SemaphoreType.DMA,
                pltpu.SemaphoreType.DMA,
            ],
            name=name)
        return f(v, d2d, z)

    return call


_scatter_lo = _make_scatter(40, "seg_scatter_lo")
_scatter_hi = _make_scatter(32, "seg_scatter_hi")


def _prep(h, ws, wd, be):
    def body(h_ref, ws_ref, wd_ref, be_ref, a_ref, b_ref):
        h_ = h_ref[...]
        a_ref[...] = jnp.dot(h_, ws_ref[...],
                             preferred_element_type=jnp.float32) + be_ref[...]
        b_ref[...] = jnp.dot(h_, wd_ref[...],
                             preferred_element_type=jnp.float32)

    grid = (N // TN,)
    big = pl.BlockSpec((TN, DH), lambda i: (i, 0))
    wsp = pl.BlockSpec((DH, DH), lambda i: (0, 0))
    bsp = pl.BlockSpec((1, DH), lambda i: (0, 0))
    return pl.pallas_call(
        body,
        grid=grid,
        in_specs=[big, wsp, wsp, bsp],
        out_specs=[big, big],
        out_shape=[jax.ShapeDtypeStruct((N, DH), jnp.float32)] * 2,
    )(h, ws, wd, be)


def _edge_mlp(z, rel, ea, wea, wdist, we2, be2, wp1, bp1, wp2, bp2):
    def body(z_ref, r_ref, e_ref, wea_ref, wd_ref, we2_ref, be2_ref,
             wp1_ref, bp1_ref, wp2_ref, bp2_ref, lo_ref, hi_ref):
        rel_ = r_ref[...]
        dist = jnp.sum(rel_ * rel_, axis=1, keepdims=True)
        pre1 = (z_ref[...] + dist * wd_ref[...]
                + jnp.dot(e_ref[...], wea_ref[...],
                          preferred_element_type=jnp.float32))
        m1 = _silu(pre1)
        m2 = _silu(jnp.dot(m1, we2_ref[...],
                           preferred_element_type=jnp.float32) + be2_ref[...])
        p = _silu(jnp.dot(m2, wp1_ref[...],
                          preferred_element_type=jnp.float32) + bp1_ref[...])
        coef = jnp.sum(p * wp2_ref[...], axis=1, keepdims=True) + bp2_ref[...]
        lo_ref[...] = jnp.concatenate(
            [m2[:, 0:32], rel_[:, 0:4] * coef,
             jnp.ones((TE, 4), jnp.float32)], axis=1)
        hi_ref[...] = m2[:, 32:64]

    grid = (EPAD // TE,)
    zsp = pl.BlockSpec((TE, DH), lambda i: (i, 0))
    rsp = pl.BlockSpec((TE, 16), lambda i: (i, 0))
    esp = pl.BlockSpec((TE, DE), lambda i: (i, 0))
    w16 = pl.BlockSpec((DE, DH), lambda i: (0, 0))
    w64 = pl.BlockSpec((DH, DH), lambda i: (0, 0))
    row = pl.BlockSpec((1, DH), lambda i: (0, 0))
    sca = pl.BlockSpec((1, 1), lambda i: (0, 0))
    losp = pl.BlockSpec((TE, 40), lambda i: (i, 0))
    hisp = pl.BlockSpec((TE, 32), lambda i: (i, 0))
    return pl.pallas_call(
        body,
        grid=grid,
        in_specs=[zsp, rsp, esp, w16, row, w64, row, w64, row, row, sca],
        out_specs=[losp, hisp],
        out_shape=[jax.ShapeDtypeStruct((EPAD, 40), jnp.float32),
                   jax.ShapeDtypeStruct((EPAD, 32), jnp.float32)],
    )(z, rel, ea, wea, wdist, we2, be2, wp1, bp1, wp2, bp2)


def _node0(h, cadd, agglo, agghi, wn1h, wn1lo, wn1hi, bn1, wn2, bn2,
           wes, wed, ben):
    def body(h_ref, c_ref, glo_ref, ghi_ref, w1h_ref, w1lo_ref, w1hi_ref,
             b1_ref, w2_ref, b2_ref,
             wes_ref, wed_ref, ben_ref, y_ref, a_ref, b_ref):
        t = _silu(jnp.dot(h_ref[...], w1h_ref[...],
                          preferred_element_type=jnp.float32)
                  + jnp.dot(glo_ref[...][:, 0:32], w1lo_ref[...],
                            preferred_element_type=jnp.float32)
                  + jnp.dot(ghi_ref[...], w1hi_ref[...],
                            preferred_element_type=jnp.float32)
                  + b1_ref[...])
        d = jnp.dot(t, w2_ref[...],
                    preferred_element_type=jnp.float32) + b2_ref[...]
        y = c_ref[...] + d
        y_ref[...] = y
        a_ref[...] = jnp.dot(y, wes_ref[...],
                             preferred_element_type=jnp.float32) + ben_ref[...]
        b_ref[...] = jnp.dot(y, wed_ref[...],
                             preferred_element_type=jnp.float32)

    grid = (N // TN,)
    big = pl.BlockSpec((TN, DH), lambda i: (i, 0))
    glo = pl.BlockSpec((TN, 40), lambda i: (i, 0))
    ghi = pl.BlockSpec((TN, 32), lambda i: (i, 0))
    w64 = pl.BlockSpec((DH, DH), lambda i: (0, 0))
    w32 = pl.BlockSpec((32, DH), lambda i: (0, 0))
    row = pl.BlockSpec((1, DH), lambda i: (0, 0))
    return pl.pallas_call(
        body,
        grid=grid,
        in_specs=[big, big, glo, ghi, w64, w32, w32, row, w64, row,
                  w64, w64, row],
        out_specs=[big, big, big],
        out_shape=[jax.ShapeDtypeStruct((N, DH), jnp.float32)] * 3,
    )(h, cadd, agglo, agghi, wn1h, wn1lo, wn1hi, bn1, wn2, bn2,
      wes, wed, ben)


def _node1(h, cadd, agglo, agghi, wn1h, wn1lo, wn1hi, bn1, wn2, bn2,
           tlo0, pa8, pb8):
    def body(h_ref, c_ref, glo_ref, ghi_ref, w1h_ref, w1lo_ref, w1hi_ref,
             b1_ref, w2_ref, b2_ref,
             t0_ref, pa_ref, pb_ref, y_ref, pc_ref):
        glo = glo_ref[...]
        t = _silu(jnp.dot(h_ref[...], w1h_ref[...],
                          preferred_element_type=jnp.float32)
                  + jnp.dot(glo[:, 0:32], w1lo_ref[...],
                            preferred_element_type=jnp.float32)
                  + jnp.dot(ghi_ref[...], w1hi_ref[...],
                            preferred_element_type=jnp.float32)
                  + b1_ref[...])
        d = jnp.dot(t, w2_ref[...],
                    preferred_element_type=jnp.float32) + b2_ref[...]
        y_ref[...] = c_ref[...] + d
        t0 = t0_ref[...][:, 32:40]
        t1 = glo[:, 32:40]
        deg = t0[:, 4:5]
        recip = 1.0 / jnp.maximum(deg, 1.0)
        pc_ref[...] = (0.5 * (pa_ref[...] + pb_ref[...])
                       + 0.25 * (t0 + t1) * recip)

    grid = (N // TN,)
    big = pl.BlockSpec((TN, DH), lambda i: (i, 0))
    glo = pl.BlockSpec((TN, 40), lambda i: (i, 0))
    ghi = pl.BlockSpec((TN, 32), lambda i: (i, 0))
    w64 = pl.BlockSpec((DH, DH), lambda i: (0, 0))
    w32 = pl.BlockSpec((32, DH), lambda i: (0, 0))
    row = pl.BlockSpec((1, DH), lambda i: (0, 0))
    td8 = pl.BlockSpec((TN, 8), lambda i: (i, 0))
    return pl.pallas_call(
        body,
        grid=grid,
        in_specs=[big, big, glo, ghi, w64, w32, w32, row, w64, row,
                  glo, td8, td8],
        out_specs=[big, td8],
        out_shape=[jax.ShapeDtypeStruct((N, DH), jnp.float32),
                   jax.ShapeDtypeStruct((N, 8), jnp.float32)],
    )(h, cadd, agglo, agghi, wn1h, wn1lo, wn1hi, bn1, wn2, bn2,
      tlo0, pa8, pb8)


def kernel(x, pos_a, pos_b, edge_index, edge_attr, We1, be1, We2, be2,
           Wp1, bp1, Wp2, bp2, Wn1, bn1, Wn2, bn2):
    f32 = jnp.float32
    src = edge_index[0]
    dst = edge_index[1]
    npad = EPAD - E

    pad_idx = (jnp.arange(npad, dtype=jnp.int32) * 97) % N
    src_g = jnp.concatenate([src, pad_idx]).reshape(IDXR, SB)
    dst_g = jnp.concatenate([dst, pad_idx]).reshape(IDXR, SB)
    dst_s = jnp.concatenate(
        [dst, jnp.full((npad,), N, jnp.int32)]).reshape(IDXR, SB)

    ea_p = jnp.pad(edge_attr, ((0, npad), (0, 0)))
    pos16 = jnp.pad(pos_a, ((0, 0), (0, 13)))
    pa8 = jnp.pad(pos_a, ((0, 0), (0, 5)))
    pb8 = jnp.pad(pos_b, ((0, 0), (0, 5)))
    z40 = jnp.zeros((SB, 40), f32)
    z32 = jnp.zeros((SB, 32), f32)

    c0 = x[:, :DH]
    c1 = x[:, DH:]

    def wsplit(i):
        w1 = We1[i]
        return (w1[0:DH], w1[DH:2 * DH], w1[2 * DH].reshape(1, DH),
                w1[2 * DH + 1:].reshape(DE, DH), be1[i].reshape(1, DH))

    ws0, wd0, wdist0, wea0, be10 = wsplit(0)
    ws1, wd1, wdist1, wea1, be11 = wsplit(1)

    rel = _rel_gather(pos16, src_g, dst_g)

    a0, b0 = _prep(c1, ws0, wd0, be10)
    z0 = _z_gather(a0, b0, src_g, dst_g)
    mlo0, mhi0 = _edge_mlp(z0, rel, ea_p, wea0, wdist0,
                         We2[0], be2[0].reshape(1, DH),
                         Wp1[0], bp1[0].reshape(1, DH),
                         Wp2[0].reshape(1, DH), bp2[0].reshape(1, 1))
    agglo0 = _scatter_lo(mlo0, dst_s, z40)
    agghi0 = _scatter_hi(mhi0, dst_s, z32)
    y0, a1, b1 = _node0(c1, c0, agglo0, agghi0,
                        Wn1[0][0:DH], Wn1[0][DH:DH + 32], Wn1[0][DH + 32:],
                        bn1[0].reshape(1, DH),
                        Wn2[0], bn2[0].reshape(1, DH), ws1, wd1, be11)

    z1 = _z_gather(a1, b1, src_g, dst_g)
    mlo1, mhi1 = _edge_mlp(z1, rel, ea_p, wea1, wdist1,
                         We2[1], be2[1].reshape(1, DH),
                         Wp1[1], bp1[1].reshape(1, DH),
                         Wp2[1].reshape(1, DH), bp2[1].reshape(1, 1))
    agglo1 = _scatter_lo(mlo1, dst_s, z40)
    agghi1 = _scatter_hi(mhi1, dst_s, z32)
    y1, pc8 = _node1(y0, c1, agglo1, agghi1,
                     Wn1[1][0:DH], Wn1[1][DH:DH + 32], Wn1[1][DH + 32:],
                     bn1[1].reshape(1, DH),
                     Wn2[1], bn2[1].reshape(1, DH),
                     agglo0, pa8, pb8)

    y = jnp.concatenate([y0, y1], axis=-1)
    pos_c = pc8[:, :3]
    return (y, pos_c, pos_a)


# speedup vs baseline: 1.2038x; 1.1582x over previous
import functools

import jax
import jax.numpy as jnp
from jax import lax
from jax.experimental import pallas as pl
from jax.experimental.pallas import tpu as pltpu
from jax.experimental.pallas import tpu_sc as plsc

N = 50000
E = 800000
D = 128
DH = 64
DE = 16

NCORE = 2
NSUB = 16
NWORK = NCORE * NSUB

CH = 512
SB = 128
NSB = CH // SB
EPW = 25088
EPAD = EPW * NWORK
TRIPS = EPW // CH
EPT = EPAD // NSUB
STRIPS = EPT // CH
IDXR = EPAD // SB

NH = N // 2
ACC = 25088
NZCH = ACC // SB
NWCH = 196

TE = 4096
TN = 2000


def _silu(v):
    return v * jax.nn.sigmoid(v)


def _sc_mesh():
    return plsc.VectorSubcoreMesh(
        core_axis_name="c", subcore_axis_name="s",
        num_cores=NCORE, num_subcores=NSUB)


def _rel_gather(pos16, s2d, d2d):
    def body(p_hbm, s_hbm, d_hbm, rel_hbm, sidx, didx, pbuf, qbuf,
             semi, semg):
        c = lax.axis_index("c")
        s = lax.axis_index("s")
        w = s * NCORE + c

        def trip(t, cy):
            g = w * EPW + t * CH
            o = g % TE
            prow = (g // TE) * (TE // 8) + (o % (TE // 8))
            pcol = 16 * (o // (TE // 8))
            r = w * (EPW // SB) + t * NSB
            di = pltpu.async_copy(s_hbm.at[pl.ds(r, NSB)], sidx, semi)
            dj = pltpu.async_copy(d_hbm.at[pl.ds(r, NSB)], didx, semi)
            di.wait()
            dj.wait()
            ds_ = []
            for j in range(NSB):
                ds_.append(pltpu.async_copy(p_hbm.at[sidx.at[j]],
                                            pbuf.at[pl.ds(j * SB, SB)],
                                            semg))
                ds_.append(pltpu.async_copy(p_hbm.at[didx.at[j]],
                                            qbuf.at[pl.ds(j * SB, SB)],
                                            semg))
            for dd in ds_:
                dd.wait()

            def sub_row(rr, cy2):
                pbuf[rr, pl.ds(0, 16)] = (pbuf[rr, pl.ds(0, 16)]
                                          - qbuf[rr, pl.ds(0, 16)])
                return cy2

            lax.fori_loop(0, CH, sub_row, 0)
            pltpu.sync_copy(pbuf,
                            rel_hbm.at[pl.ds(prow, CH), pl.ds(pcol, 16)])
            return cy

        lax.fori_loop(0, TRIPS, trip, 0)

    f = pl.kernel(
        body,
        out_type=jax.ShapeDtypeStruct((EPAD // 8, 128), jnp.float32),
        mesh=_sc_mesh(),
        compiler_params=pltpu.CompilerParams(use_tc_tiling_on_sc=False),
        scratch_types=[
            pltpu.VMEM((NSB, SB), jnp.int32),
            pltpu.VMEM((NSB, SB), jnp.int32),
            pltpu.VMEM((CH, 16), jnp.float32),
            pltpu.VMEM((CH, 16), jnp.float32),
            pltpu.SemaphoreType.DMA,
            pltpu.SemaphoreType.DMA,
        ],
        name="rel_gather")
    return f(pos16, s2d, d2d)


def _z_gather(a_tab, b_tab, s2d, d2d):
    def body(a_hbm, b_hbm, s_hbm, d_hbm, z_hbm, sidx, didx, abuf, bbuf,
             semi, semg):
        c = lax.axis_index("c")
        s = lax.axis_index("s")
        w = s * NCORE + c

        def trip(t, cy):
            g = w * EPW + t * CH
            o = g % TE
            prow = (g // TE) * (TE // 2) + (o % (TE // 2))
            pcol = DH * (o // (TE // 2))
            r = w * (EPW // SB) + t * NSB
            di = pltpu.async_copy(s_hbm.at[pl.ds(r, NSB)], sidx, semi)
            dj = pltpu.async_copy(d_hbm.at[pl.ds(r, NSB)], didx, semi)
            di.wait()
            dj.wait()
            ds_ = []
            for j in range(NSB):
                ds_.append(pltpu.async_copy(a_hbm.at[sidx.at[j]],
                                            abuf.at[pl.ds(j * SB, SB)],
                                            semg))
                ds_.append(pltpu.async_copy(b_hbm.at[didx.at[j]],
                                            bbuf.at[pl.ds(j * SB, SB)],
                                            semg))
            for dd in ds_:
                dd.wait()

            def add_row(rr, cy2):
                for j4 in range(DH // 16):
                    abuf[rr, pl.ds(j4 * 16, 16)] = (
                        abuf[rr, pl.ds(j4 * 16, 16)]
                        + bbuf[rr, pl.ds(j4 * 16, 16)])
                return cy2

            lax.fori_loop(0, CH, add_row, 0)
            pltpu.sync_copy(abuf,
                            z_hbm.at[pl.ds(prow, CH), pl.ds(pcol, DH)])
            return cy

        lax.fori_loop(0, TRIPS, trip, 0)

    f = pl.kernel(
        body,
        out_type=jax.ShapeDtypeStruct((EPAD // 2, 128), jnp.float32),
        mesh=_sc_mesh(),
        compiler_params=pltpu.CompilerParams(use_tc_tiling_on_sc=False),
        scratch_types=[
            pltpu.VMEM((NSB, SB), jnp.int32),
            pltpu.VMEM((NSB, SB), jnp.int32),
            pltpu.VMEM((CH, DH), jnp.float32),
            pltpu.VMEM((CH, DH), jnp.float32),
            pltpu.SemaphoreType.DMA,
            pltpu.SemaphoreType.DMA,
        ],
        name="z_gather")
    return f(a_tab, b_tab, s2d, d2d)


def _scatter_body_common(s, c):
    return c * NH, NH + 4 * s


def _zero_phase(z_hbm, acc, s):
    for j in range(13):
        cidx = s + NSUB * j

        @pl.when(cidx < NZCH)
        def _():
            pltpu.sync_copy(z_hbm, acc.at[pl.ds(cidx * SB, SB)])


def _writeout_phase(acc, out_hbm, s, base):
    for j in range(13):
        cidx = s + NSUB * j

        @pl.when(cidx < NWCH - 1)
        def _():
            pltpu.sync_copy(acc.at[pl.ds(cidx * SB, SB)],
                            out_hbm.at[pl.ds(base + cidx * SB, SB)])

        @pl.when(cidx == NWCH - 1)
        def _():
            pltpu.sync_copy(acc.at[pl.ds(cidx * SB, 40)],
                            out_hbm.at[pl.ds(base + cidx * SB, 40)])


def _adjust_idx(didx, base, dump):
    for j in range(NSB):
        for v in range(SB // 16):
            dv = didx[j, pl.ds(v * 16, 16)]
            loc = dv - base
            ok = (loc >= 0) & (loc < NH)
            didx[j, pl.ds(v * 16, 16)] = jnp.where(ok, loc, dump)


def _scatter_mtd(m, td, d2d, z32, z8):
    def body(m_hbm, t_hbm, d_hbm, z32_hbm, z8_hbm, om_hbm, ot_hbm,
             didx, mbuf, tbuf, accm, acct, semi, sems):
        c = lax.axis_index("c")
        s = lax.axis_index("s")
        base, dump = _scatter_body_common(s, c)
        _zero_phase(z32_hbm, accm, s)
        _zero_phase(z8_hbm, acct, s)
        plsc.subcore_barrier()

        def trip(t, cy):
            g = s * EPT + t * CH
            i = g // TE
            o = g % TE
            mrow = i * (TE // 4) + (o % (TE // 4))
            mcol = 32 * (o // (TE // 4))
            trow = i * (TE // 16)
            j0 = o // (TE // 16)
            r = s * (EPT // SB) + t * NSB
            di = pltpu.async_copy(d_hbm.at[pl.ds(r, NSB)], didx, semi)
            dm = pltpu.async_copy(
                m_hbm.at[pl.ds(mrow, CH), pl.ds(mcol, 32)], mbuf, semi)
            dt0 = pltpu.async_copy(
                t_hbm.at[pl.ds(trow, TE // 16), pl.ds(8 * j0, 8)],
                tbuf.at[pl.ds(0, TE // 16)], semi)
            dt1 = pltpu.async_copy(
                t_hbm.at[pl.ds(trow, TE // 16), pl.ds(8 * (j0 + 1), 8)],
                tbuf.at[pl.ds(TE // 16, TE // 16)], semi)
            di.wait()
            dm.wait()
            dt0.wait()
            dt1.wait()
            _adjust_idx(didx, base, dump)
            ds_ = []
            for j in range(NSB):
                ds_.append(pltpu.async_copy(mbuf.at[pl.ds(j * SB, SB)],
                                            accm.at[didx.at[j]], sems,
                                            add=True))
                ds_.append(pltpu.async_copy(tbuf.at[pl.ds(j * SB, SB)],
                                            acct.at[didx.at[j]], sems,
                                            add=True))
            for dd in ds_:
                dd.wait()
            return cy

        lax.fori_loop(0, STRIPS, trip, 0)
        plsc.subcore_barrier()
        _writeout_phase(accm, om_hbm, s, base)
        _writeout_phase(acct, ot_hbm, s, base)

    f = pl.kernel(
        body,
        out_type=(jax.ShapeDtypeStruct((N, 32), jnp.float32),
                  jax.ShapeDtypeStruct((N, 8), jnp.float32)),
        mesh=_sc_mesh(),
        compiler_params=pltpu.CompilerParams(use_tc_tiling_on_sc=False),
        scratch_types=[
            pltpu.VMEM((NSB, SB), jnp.int32),
            pltpu.VMEM((CH, 32), jnp.float32),
            pltpu.VMEM((CH, 8), jnp.float32),
            pltpu.VMEM_SHARED((ACC, 32), jnp.float32),
            pltpu.VMEM_SHARED((ACC, 8), jnp.float32),
            pltpu.SemaphoreType.DMA,
            pltpu.SemaphoreType.DMA,
        ],
        name="seg_scatter_mtd")
    return f(m, td, d2d, z32, z8)


def _scatter_m(m, d2d, z32):
    def body(m_hbm, d_hbm, z32_hbm, om_hbm, didx, mbuf, accm, semi, sems):
        c = lax.axis_index("c")
        s = lax.axis_index("s")
        base, dump = _scatter_body_common(s, c)
        _zero_phase(z32_hbm, accm, s)
        plsc.subcore_barrier()

        def trip(t, cy):
            g = s * EPT + t * CH
            i = g // TE
            o = g % TE
            mrow = i * (TE // 4) + (o % (TE // 4))
            mcol = 32 * (o // (TE // 4))
            r = s * (EPT // SB) + t * NSB
            di = pltpu.async_copy(d_hbm.at[pl.ds(r, NSB)], didx, semi)
            dm = pltpu.async_copy(
                m_hbm.at[pl.ds(mrow, CH), pl.ds(mcol, 32)], mbuf, semi)
            di.wait()
            dm.wait()
            _adjust_idx(didx, base, dump)
            ds_ = []
            for j in range(NSB):
                ds_.append(pltpu.async_copy(mbuf.at[pl.ds(j * SB, SB)],
                                            accm.at[didx.at[j]], sems,
                                            add=True))
            for dd in ds_:
                dd.wait()
            return cy

        lax.fori_loop(0, STRIPS, trip, 0)
        plsc.subcore_barrier()
        _writeout_phase(accm, om_hbm, s, base)

    f = pl.kernel(
        body,
        out_type=jax.ShapeDtypeStruct((N, 32), jnp.float32),
        mesh=_sc_mesh(),
        compiler_params=pltpu.CompilerParams(use_tc_tiling_on_sc=False),
        scratch_types=[
            pltpu.VMEM((NSB, SB), jnp.int32),
            pltpu.VMEM((CH, 32), jnp.float32),
            pltpu.VMEM_SHARED((ACC, 32), jnp.float32),
            pltpu.SemaphoreType.DMA,
            pltpu.SemaphoreType.DMA,
        ],
        name="seg_scatter_m")
    return f(m, d2d, z32)


def _prep(h, ws, wd, be):
    def body(h_ref, ws_ref, wd_ref, be_ref, a_ref, b_ref):
        h_ = h_ref[...]
        a_ref[...] = jnp.dot(h_, ws_ref[...],
                             preferred_element_type=jnp.float32) + be_ref[...]
        b_ref[...] = jnp.dot(h_, wd_ref[...],
                             preferred_element_type=jnp.float32)

    grid = (N // TN,)
    big = pl.BlockSpec((TN, DH), lambda i: (i, 0))
    wsp = pl.BlockSpec((DH, DH), lambda i: (0, 0))
    bsp = pl.BlockSpec((1, DH), lambda i: (0, 0))
    return pl.pallas_call(
        body,
        grid=grid,
        in_specs=[big, wsp, wsp, bsp],
        out_specs=[big, big],
        out_shape=[jax.ShapeDtypeStruct((N, DH), jnp.float32)] * 2,
    )(h, ws, wd, be)


def _ea_pack(ea):
    def body(e_ref, o_ref):
        e_ = e_ref[...]
        o_ref[...] = jnp.concatenate(
            [e_[j * (TE // 8):(j + 1) * (TE // 8)] for j in range(8)], axis=1)

    grid = (EPAD // TE,)
    return pl.pallas_call(
        body,
        grid=grid,
        in_specs=[pl.BlockSpec((TE, DE), lambda i: (i, 0))],
        out_specs=[pl.BlockSpec((TE // 8, 128), lambda i: (i, 0))],
        out_shape=[jax.ShapeDtypeStruct((EPAD // 8, 128), jnp.float32)],
    )(ea)[0]


def _edge_mlp(z, rel, ea, wea, wdist, we2, be2, wp1, bp1, wp2, bp2):
    def body(z_ref, r_ref, e_ref, wea_ref, wd_ref, we2_ref, be2_ref,
             wp1_ref, bp1_ref, wp2_ref, bp2_ref, lo_ref, hi_ref, td_ref):
        z2 = z_ref[...]
        z_ = jnp.concatenate([z2[:, 0:DH], z2[:, DH:128]], axis=0)
        r8 = r_ref[...]
        rel_ = jnp.concatenate(
            [r8[:, 16 * j:16 * (j + 1)] for j in range(8)], axis=0)
        e8 = e_ref[...]
        ea_ = jnp.concatenate(
            [e8[:, 16 * j:16 * (j + 1)] for j in range(8)], axis=0)
        dist = jnp.sum(rel_ * rel_, axis=1, keepdims=True)
        pre1 = (z_ + dist * wd_ref[...]
                + jnp.dot(ea_, wea_ref[...],
                          preferred_element_type=jnp.float32))
        m1 = _silu(pre1)
        m2 = _silu(jnp.dot(m1, we2_ref[...],
                           preferred_element_type=jnp.float32) + be2_ref[...])
        p = _silu(jnp.dot(m2, wp1_ref[...],
                          preferred_element_type=jnp.float32) + bp1_ref[...])
        coef = jnp.sum(p * wp2_ref[...], axis=1, keepdims=True) + bp2_ref[...]
        td = jnp.concatenate(
            [rel_[:, 0:4] * coef, jnp.ones((TE, 4), jnp.float32)], axis=1)
        q = TE // 4
        lo_ref[...] = jnp.concatenate(
            [m2[j * q:(j + 1) * q, 0:32] for j in range(4)], axis=1)
        hi_ref[...] = jnp.concatenate(
            [m2[j * q:(j + 1) * q, 32:64] for j in range(4)], axis=1)
        w = TE // 16
        td_ref[...] = jnp.concatenate(
            [td[j * w:(j + 1) * w] for j in range(16)], axis=1)

    grid = (EPAD // TE,)
    zsp = pl.BlockSpec((TE // 2, 128), lambda i: (i, 0))
    rsp = pl.BlockSpec((TE // 8, 128), lambda i: (i, 0))
    w16 = pl.BlockSpec((DE, DH), lambda i: (0, 0))
    w64 = pl.BlockSpec((DH, DH), lambda i: (0, 0))
    row = pl.BlockSpec((1, DH), lambda i: (0, 0))
    sca = pl.BlockSpec((1, 1), lambda i: (0, 0))
    msp = pl.BlockSpec((TE // 4, 128), lambda i: (i, 0))
    tsp = pl.BlockSpec((TE // 16, 128), lambda i: (i, 0))
    return pl.pallas_call(
        body,
        grid=grid,
        in_specs=[zsp, rsp, rsp, w16, row, w64, row, w64, row, row, sca],
        out_specs=[msp, msp, tsp],
        out_shape=[jax.ShapeDtypeStruct((EPAD // 4, 128), jnp.float32),
                   jax.ShapeDtypeStruct((EPAD // 4, 128), jnp.float32),
                   jax.ShapeDtypeStruct((EPAD // 16, 128), jnp.float32)],
    )(z, rel, ea, wea, wdist, we2, be2, wp1, bp1, wp2, bp2)


def _node0(h, cadd, agglo, agghi, wn1h, wn1lo, wn1hi, bn1, wn2, bn2,
           wes, wed, ben):
    def body(h_ref, c_ref, glo_ref, ghi_ref, w1h_ref, w1lo_ref, w1hi_ref,
             b1_ref, w2_ref, b2_ref,
             wes_ref, wed_ref, ben_ref, y_ref, a_ref, b_ref):
        t = _silu(jnp.dot(h_ref[...], w1h_ref[...],
                          preferred_element_type=jnp.float32)
                  + jnp.dot(glo_ref[...], w1lo_ref[...],
                            preferred_element_type=jnp.float32)
                  + jnp.dot(ghi_ref[...], w1hi_ref[...],
                            preferred_element_type=jnp.float32)
                  + b1_ref[...])
        d = jnp.dot(t, w2_ref[...],
                    preferred_element_type=jnp.float32) + b2_ref[...]
        y = c_ref[...] + d
        y_ref[...] = y
        a_ref[...] = jnp.dot(y, wes_ref[...],
                             preferred_element_type=jnp.float32) + ben_ref[...]
        b_ref[...] = jnp.dot(y, wed_ref[...],
                             preferred_element_type=jnp.float32)

    grid = (N // TN,)
    big = pl.BlockSpec((TN, DH), lambda i: (i, 0))
    glo = pl.BlockSpec((TN, 32), lambda i: (i, 0))
    ghi = pl.BlockSpec((TN, 32), lambda i: (i, 0))
    w64 = pl.BlockSpec((DH, DH), lambda i: (0, 0))
    w32 = pl.BlockSpec((32, DH), lambda i: (0, 0))
    row = pl.BlockSpec((1, DH), lambda i: (0, 0))
    return pl.pallas_call(
        body,
        grid=grid,
        in_specs=[big, big, glo, ghi, w64, w32, w32, row, w64, row,
                  w64, w64, row],
        out_specs=[big, big, big],
        out_shape=[jax.ShapeDtypeStruct((N, DH), jnp.float32)] * 3,
    )(h, cadd, agglo, agghi, wn1h, wn1lo, wn1hi, bn1, wn2, bn2,
      wes, wed, ben)


def _node1(h, cadd, agglo, agghi, wn1h, wn1lo, wn1hi, bn1, wn2, bn2,
           td0, td1, pa8, pb8):
    def body(h_ref, c_ref, glo_ref, ghi_ref, w1h_ref, w1lo_ref, w1hi_ref,
             b1_ref, w2_ref, b2_ref,
             t0_ref, t1_ref, pa_ref, pb_ref, y_ref, pc_ref):
        t = _silu(jnp.dot(h_ref[...], w1h_ref[...],
                          preferred_element_type=jnp.float32)
                  + jnp.dot(glo_ref[...], w1lo_ref[...],
                            preferred_element_type=jnp.float32)
                  + jnp.dot(ghi_ref[...], w1hi_ref[...],
                            preferred_element_type=jnp.float32)
                  + b1_ref[...])
        d = jnp.dot(t, w2_ref[...],
                    preferred_element_type=jnp.float32) + b2_ref[...]
        y_ref[...] = c_ref[...] + d
        t0 = t0_ref[...]
        t1 = t1_ref[...]
        deg = t0[:, 4:5]
        recip = 1.0 / jnp.maximum(deg, 1.0)
        pc_ref[...] = (0.5 * (pa_ref[...] + pb_ref[...])
                       + 0.25 * (t0 + t1) * recip)

    grid = (N // TN,)
    big = pl.BlockSpec((TN, DH), lambda i: (i, 0))
    glo = pl.BlockSpec((TN, 32), lambda i: (i, 0))
    ghi = pl.BlockSpec((TN, 32), lambda i: (i, 0))
    w64 = pl.BlockSpec((DH, DH), lambda i: (0, 0))
    w32 = pl.BlockSpec((32, DH), lambda i: (0, 0))
    row = pl.BlockSpec((1, DH), lambda i: (0, 0))
    td8 = pl.BlockSpec((TN, 8), lambda i: (i, 0))
    return pl.pallas_call(
        body,
        grid=grid,
        in_specs=[big, big, glo, ghi, w64, w32, w32, row, w64, row,
                  td8, td8, td8, td8],
        out_specs=[big, td8],
        out_shape=[jax.ShapeDtypeStruct((N, DH), jnp.float32),
                   jax.ShapeDtypeStruct((N, 8), jnp.float32)],
    )(h, cadd, agglo, agghi, wn1h, wn1lo, wn1hi, bn1, wn2, bn2,
      td0, td1, pa8, pb8)


def kernel(x, pos_a, pos_b, edge_index, edge_attr, We1, be1, We2, be2,
           Wp1, bp1, Wp2, bp2, Wn1, bn1, Wn2, bn2):
    f32 = jnp.float32
    src = edge_index[0]
    dst = edge_index[1]
    npad = EPAD - E

    pad_idx = (jnp.arange(npad, dtype=jnp.int32) * 97) % N
    src_g = jnp.concatenate([src, pad_idx]).reshape(IDXR, SB)
    dst_g = jnp.concatenate([dst, pad_idx]).reshape(IDXR, SB)
    dst_s = jnp.concatenate(
        [dst, jnp.full((npad,), N, jnp.int32)]).reshape(IDXR, SB)

    pos16 = jnp.pad(pos_a, ((0, 0), (0, 13)))
    pa8 = jnp.pad(pos_a, ((0, 0), (0, 5)))
    pb8 = jnp.pad(pos_b, ((0, 0), (0, 5)))
    z32 = jnp.zeros((SB, 32), f32)
    z8 = jnp.zeros((SB, 8), f32)

    c0 = x[:, :DH]
    c1 = x[:, DH:]

    def wsplit(i):
        w1 = We1[i]
        return (w1[0:DH], w1[DH:2 * DH], w1[2 * DH].reshape(1, DH),
                w1[2 * DH + 1:].reshape(DE, DH), be1[i].reshape(1, DH))

    ws0, wd0, wdist0, wea0, be10 = wsplit(0)
    ws1, wd1, wdist1, wea1, be11 = wsplit(1)

    rel = _rel_gather(pos16, src_g, dst_g)
    eap = _ea_pack(edge_attr)

    a0, b0 = _prep(c1, ws0, wd0, be10)
    z0 = _z_gather(a0, b0, src_g, dst_g)
    mlo0, mhi0, tde0 = _edge_mlp(z0, rel, eap, wea0, wdist0,
                         We2[0], be2[0].reshape(1, DH),
                         Wp1[0], bp1[0].reshape(1, DH),
                         Wp2[0].reshape(1, DH), bp2[0].reshape(1, 1))
    agglo0, aggt0 = _scatter_mtd(mlo0, tde0, dst_s, z32, z8)
    agghi0 = _scatter_m(mhi0, dst_s, z32)
    y0, a1, b1 = _node0(c1, c0, agglo0, agghi0,
                        Wn1[0][0:DH], Wn1[0][DH:DH + 32], Wn1[0][DH + 32:],
                        bn1[0].reshape(1, DH),
                        Wn2[0], bn2[0].reshape(1, DH), ws1, wd1, be11)

    z1 = _z_gather(a1, b1, src_g, dst_g)
    mlo1, mhi1, tde1 = _edge_mlp(z1, rel, eap, wea1, wdist1,
                         We2[1], be2[1].reshape(1, DH),
                         Wp1[1], bp1[1].reshape(1, DH),
                         Wp2[1].reshape(1, DH), bp2[1].reshape(1, 1))
    agglo1, aggt1 = _scatter_mtd(mlo1, tde1, dst_s, z32, z8)
    agghi1 = _scatter_m(mhi1, dst_s, z32)
    y1, pc8 = _node1(y0, c1, agglo1, agghi1,
                     Wn1[1][0:DH], Wn1[1][DH:DH + 32], Wn1[1][DH + 32:],
                     bn1[1].reshape(1, DH),
                     Wn2[1], bn2[1].reshape(1, DH),
                     aggt0, aggt1, pa8, pb8)

    y = jnp.concatenate([y0, y1], axis=-1)
    pos_c = pc8[:, :3]
    return (y, pos_c, pos_a)
